# Initial kernel scaffold; baseline (speedup 1.0000x reference)
import jax
import jax.numpy as jnp
from jax.experimental import pallas as pl


def kernel(Q, Q_ok, td_refs, td_mask, td_node_state, W_res, W_k, node_embed, b_o):
    raise NotImplementedError("write your pallas kernel here")


# speedup vs baseline: 1.2911x; 1.2911x over previous
import functools

import jax
import jax.numpy as jnp
from jax.experimental import pallas as pl
from jax.experimental.pallas import tpu as pltpu

B, DK, SEQ_DIM, HID_DIM, N, NE_DIM = 32, 512, 4096, 512, 2048, 32
NB = 8
BN = N // NB
INV_SQRT_S = 1.0 / (float(SEQ_DIM) ** 0.5)


def _unpack_bitmajor(x_u8, out_dtype):
    xi = x_u8.astype(jnp.int32)
    pieces = [((xi >> (7 - j)) & 1).astype(out_dtype) for j in range(8)]
    return jnp.concatenate(pieces, axis=1)


def _seqnet_kernel(q_ref, qok_ref, refs_ref, mask_ref, nst_ref,
                   wres_ref, wk_ref, ne_ref, bo_ref, out_ref,
                   a_s, scores_s, g_s):
    i = pl.program_id(0)

    @pl.when(i == 0)
    def _prologue():
        qu = _unpack_bitmajor(q_ref[...], jnp.float32)
        qok = _unpack_bitmajor(qok_ref[...], jnp.float32)
        a_s[0:B, :] = qu * qok * INV_SQRT_S
        q_proj = jnp.dot(qu, wres_ref[...],
                         preferred_element_type=jnp.float32)
        v = jax.lax.dot_general(q_proj, wk_ref[...],
                                (((1,), (1,)), ((), ())),
                                preferred_element_type=jnp.float32)
        a_s[B:2 * B, :] = v

    kb = _unpack_bitmajor(refs_ref[...], jnp.float32)
    s2 = jax.lax.dot_general(a_s[...], kb, (((1,), (1,)), ((), ())),
                             preferred_element_type=jnp.float32)
    scores_s[:, pl.ds(i * BN, BN)] = s2[0:B, :]
    g_s[:, pl.ds(i * BN, BN)] = s2[B:2 * B, :]

    @pl.when(i == NB - 1)
    def _epilogue():
        s = jnp.where(mask_ref[...] > 0, scores_s[...], -1e9)
        m = jnp.max(s, axis=1, keepdims=True)
        e = jnp.exp(s - m)
        w = e / jnp.sum(e, axis=1, keepdims=True)
        ne0 = jnp.sum(ne_ref[0:1, :], keepdims=True)
        ne1 = jnp.sum(ne_ref[1:2, :], keepdims=True)
        c = ne0 * nst_ref[0:1, :] + ne1 * nst_ref[1:2, :]
        out_ref[...] = w * g_s[...] + c + bo_ref[...]


@functools.partial(jax.jit, static_argnames=())
def kernel(Q, Q_ok, td_refs, td_mask, td_node_state, W_res, W_k, node_embed, b_o):
    wres_p = W_res.reshape(DK, 8, HID_DIM).transpose(1, 0, 2).reshape(SEQ_DIM, HID_DIM)
    wk_p = W_k.reshape(DK, 8, HID_DIM).transpose(1, 0, 2).reshape(SEQ_DIM, HID_DIM)
    nst_t = td_node_state.T
    mask_f = td_mask.astype(jnp.float32)

    full = lambda shape: pl.BlockSpec(shape, lambda i: (0,) * len(shape))
    out = pl.pallas_call(
        _seqnet_kernel,
        grid=(NB,),
        in_specs=[
            full((B, DK)),
            full((B, DK)),
            pl.BlockSpec((BN, DK), lambda i: (i, 0)),
            full((B, N)),
            full((2, N)),
            full((SEQ_DIM, HID_DIM)),
            full((SEQ_DIM, HID_DIM)),
            full((2, NE_DIM)),
            full((1, N)),
        ],
        out_specs=full((B, N)),
        out_shape=jax.ShapeDtypeStruct((B, N), jnp.float32),
        scratch_shapes=[
            pltpu.VMEM((2 * B, SEQ_DIM), jnp.float32),
            pltpu.VMEM((B, N), jnp.float32),
            pltpu.VMEM((B, N), jnp.float32),
        ],
    )(Q, Q_ok, td_refs, mask_f, nst_t, wres_p, wk_p, node_embed, b_o)
    return out


# speedup vs baseline: 3.3441x; 2.5901x over previous
import jax
import jax.numpy as jnp
from jax.experimental import pallas as pl
from jax.experimental.pallas import tpu as pltpu

B, DK, SEQ_DIM, HID_DIM, N, NE_DIM = 32, 512, 4096, 512, 2048, 32
NB = 8
BN = N // NB
INV_SQRT_S = 1.0 / (float(SEQ_DIM) ** 0.5)


def _bitplane(xi, j, out_dtype):
    return ((xi >> (7 - j)) & 1).astype(out_dtype)


def _unpack_bitmajor(x_u8, out_dtype):
    xi = x_u8.astype(jnp.int32)
    return jnp.concatenate([_bitplane(xi, j, out_dtype) for j in range(8)], axis=1)


def _seqnet_kernel(q_ref, qok_ref, refs_ref, mask_ref, nst_ref,
                   wres_ref, wk_ref, ne_ref, bo_ref, out_ref,
                   a_s, scores_s, g_s):
    i = pl.program_id(0)

    @pl.when(i == 0)
    def _prologue():
        qi = q_ref[...].astype(jnp.int32)
        qoki = qok_ref[...].astype(jnp.int32)
        qplanes = [_bitplane(qi, j, jnp.float32) for j in range(8)]
        for j in range(8):
            a_s[0:B, j * DK:(j + 1) * DK] = (
                qplanes[j] * _bitplane(qoki, j, jnp.float32) * INV_SQRT_S)
        q_proj = sum(
            jnp.dot(qplanes[j], wres_ref[:, j, :],
                    preferred_element_type=jnp.float32)
            for j in range(8))
        for j in range(8):
            a_s[B:2 * B, j * DK:(j + 1) * DK] = jax.lax.dot_general(
                q_proj, wk_ref[:, j, :], (((1,), (1,)), ((), ())),
                preferred_element_type=jnp.float32)

    kb = _unpack_bitmajor(refs_ref[...], jnp.float32)
    s2 = jax.lax.dot_general(a_s[...], kb, (((1,), (1,)), ((), ())),
                             preferred_element_type=jnp.float32)
    scores_s[:, pl.ds(i * BN, BN)] = s2[0:B, :]
    g_s[:, pl.ds(i * BN, BN)] = s2[B:2 * B, :]

    @pl.when(i == NB - 1)
    def _epilogue():
        s = jnp.where(mask_ref[...] > 0, scores_s[...], -1e9)
        m = jnp.max(s, axis=1, keepdims=True)
        e = jnp.exp(s - m)
        w = e / jnp.sum(e, axis=1, keepdims=True)
        ne0 = jnp.sum(ne_ref[0:1, :], keepdims=True)
        ne1 = jnp.sum(ne_ref[1:2, :], keepdims=True)
        c = ne0 * nst_ref[0:1, :] + ne1 * nst_ref[1:2, :]
        out_ref[...] = w * g_s[...] + c + bo_ref[...]


@jax.jit
def kernel(Q, Q_ok, td_refs, td_mask, td_node_state, W_res, W_k, node_embed, b_o):
    wres_3d = W_res.reshape(DK, 8, HID_DIM)
    wk_3d = W_k.reshape(DK, 8, HID_DIM)
    nst_t = td_node_state.T
    mask_f = td_mask.astype(jnp.float32)

    full = lambda shape: pl.BlockSpec(shape, lambda i: (0,) * len(shape))
    out = pl.pallas_call(
        _seqnet_kernel,
        grid=(NB,),
        in_specs=[
            full((B, DK)),
            full((B, DK)),
            pl.BlockSpec((BN, DK), lambda i: (i, 0)),
            full((B, N)),
            full((2, N)),
            full((DK, 8, HID_DIM)),
            full((DK, 8, HID_DIM)),
            full((2, NE_DIM)),
            full((1, N)),
        ],
        out_specs=full((B, N)),
        out_shape=jax.ShapeDtypeStruct((B, N), jnp.float32),
        scratch_shapes=[
            pltpu.VMEM((2 * B, SEQ_DIM), jnp.float32),
            pltpu.VMEM((B, N), jnp.float32),
            pltpu.VMEM((B, N), jnp.float32),
        ],
    )(Q, Q_ok, td_refs, mask_f, nst_t, wres_3d, wk_3d, node_embed, b_o)
    return out


# speedup vs baseline: 3.3533x; 1.0028x over previous
import jax
import jax.numpy as jnp
from jax.experimental import pallas as pl
from jax.experimental.pallas import tpu as pltpu

B, DK, SEQ_DIM, HID_DIM, N, NE_DIM = 32, 512, 4096, 512, 2048, 32
NB = 8
BN = N // NB
INV_SQRT_S = 1.0 / (float(SEQ_DIM) ** 0.5)


def _bitplane(xi, j, out_dtype):
    return ((xi >> (7 - j)) & 1).astype(out_dtype)


def _unpack_bitmajor(x_u8, out_dtype):
    xi = x_u8.astype(jnp.int32)
    return jnp.concatenate([_bitplane(xi, j, out_dtype) for j in range(8)], axis=1)


def _seqnet_kernel(q_ref, qok_ref, refs_ref, mask_ref, nst_ref,
                   wres_ref, wk_ref, ne_ref, bo_ref, out_ref,
                   a_s, scores_s, g_s):
    i = pl.program_id(0)

    @pl.when(i == 0)
    def _prologue():
        qi = q_ref[...].astype(jnp.int32)
        qoki = qok_ref[...].astype(jnp.int32)
        qplanes = [_bitplane(qi, j, jnp.float32) for j in range(8)]
        for j in range(8):
            a_s[0:B, j * DK:(j + 1) * DK] = (
                qplanes[j] * _bitplane(qoki, j, jnp.float32) * INV_SQRT_S
            ).astype(jnp.bfloat16)
        q_proj = sum(
            jnp.dot(qplanes[j], wres_ref[:, j, :],
                    preferred_element_type=jnp.float32)
            for j in range(8))
        for j in range(8):
            a_s[B:2 * B, j * DK:(j + 1) * DK] = jax.lax.dot_general(
                q_proj, wk_ref[:, j, :], (((1,), (1,)), ((), ())),
                preferred_element_type=jnp.float32).astype(jnp.bfloat16)

    kb = _unpack_bitmajor(refs_ref[...], jnp.bfloat16)
    s2 = jax.lax.dot_general(a_s[...], kb, (((1,), (1,)), ((), ())),
                             preferred_element_type=jnp.float32)
    scores_s[:, pl.ds(i * BN, BN)] = s2[0:B, :]
    g_s[:, pl.ds(i * BN, BN)] = s2[B:2 * B, :]

    @pl.when(i == NB - 1)
    def _epilogue():
        s = jnp.where(mask_ref[...] > 0, scores_s[...], -1e9)
        m = jnp.max(s, axis=1, keepdims=True)
        e = jnp.exp(s - m)
        w = e / jnp.sum(e, axis=1, keepdims=True)
        ne0 = jnp.sum(ne_ref[0:1, :], keepdims=True)
        ne1 = jnp.sum(ne_ref[1:2, :], keepdims=True)
        c = ne0 * nst_ref[0:1, :] + ne1 * nst_ref[1:2, :]
        out_ref[...] = w * g_s[...] + c + bo_ref[...]


@jax.jit
def kernel(Q, Q_ok, td_refs, td_mask, td_node_state, W_res, W_k, node_embed, b_o):
    wres_3d = W_res.reshape(DK, 8, HID_DIM)
    wk_3d = W_k.reshape(DK, 8, HID_DIM)
    nst_t = td_node_state.T
    mask_f = td_mask.astype(jnp.float32)

    full = lambda shape: pl.BlockSpec(shape, lambda i: (0,) * len(shape))
    out = pl.pallas_call(
        _seqnet_kernel,
        grid=(NB,),
        in_specs=[
            full((B, DK)),
            full((B, DK)),
            pl.BlockSpec((BN, DK), lambda i: (i, 0)),
            full((B, N)),
            full((2, N)),
            full((DK, 8, HID_DIM)),
            full((DK, 8, HID_DIM)),
            full((2, NE_DIM)),
            full((1, N)),
        ],
        out_specs=full((B, N)),
        out_shape=jax.ShapeDtypeStruct((B, N), jnp.float32),
        scratch_shapes=[
            pltpu.VMEM((2 * B, SEQ_DIM), jnp.bfloat16),
            pltpu.VMEM((B, N), jnp.float32),
            pltpu.VMEM((B, N), jnp.float32),
        ],
    )(Q, Q_ok, td_refs, mask_f, nst_t, wres_3d, wk_3d, node_embed, b_o)
    return out
